# Initial kernel scaffold; baseline (speedup 1.0000x reference)
import jax
import jax.numpy as jnp
from jax.experimental import pallas as pl


def kernel(x, edge_index, edge_attr, params):
    raise NotImplementedError("write your pallas kernel here")


# speedup vs baseline: 11.9977x; 11.9977x over previous
import functools

import jax
import jax.numpy as jnp
from jax import lax
from jax.experimental import pallas as pl
from jax.experimental.pallas import tpu as pltpu
from jax.experimental.pallas import tpu_sc as plsc

N = 10000
E = 320000
C = 128
HEADS = (4, 4, 1)

NC = 2
NS = 16
NW = NC * NS
LANES = 16

EW = E // NW
CH = 80
NCHUNK = EW // CH
NG = CH // LANES
CH2 = 80
NCHUNK2 = EW // CH2
NG2 = CH2 // LANES
RPT = 624
TAIL = N - RPT * NS
ZR = 48
KV = C // LANES

def _take16(v, ix):
    return lax.gather(
        v, ix[:, None],
        dimension_numbers=lax.GatherDimensionNumbers(
            offset_dims=(), collapsed_slice_dims=(0,), start_index_map=(0,)),
        slice_sizes=(1,),
        mode=lax.GatherScatterMode.PROMISE_IN_BOUNDS)


def _lanesum(v):
    for sh in (8, 4, 2, 1):
        v = v + _take16(v, lax.iota(jnp.int32, LANES) ^ sh)
    return v


def _bcast_idx(j):
    return jnp.full((LANES,), j, jnp.int32)


def _msg_pass_body(xl_hbm, xr_hbm, src_hbm, dst_hbm, attr_hbm, we_hbm, att_hbm,
                   acc_hbm, ea_hbm,
                   idx_s, idx_d, attr_v, xl_rows, xr_rows, ea_buf, we_v, att_v,
                   out_sh, sem1, sem2):
    c = lax.axis_index("c")
    s = lax.axis_index("s")
    wid = s * NC + c

    zero16 = jnp.zeros((LANES,), jnp.float32)
    lane_iota = lax.iota(jnp.int32, LANES)

    def zrow(r, _):
        for k in range(KV):
            xl_rows[r, pl.ds(k * LANES, LANES)] = zero16
        return 0
    lax.fori_loop(0, ZR, zrow, 0)
    for j in range(RPT // ZR):
        r0 = s * RPT + j * ZR
        pltpu.sync_copy(xl_rows.at[pl.ds(0, ZR)], out_sh.at[pl.ds(r0, ZR)])

    @pl.when(s == NS - 1)
    def _zero_tail():
        pltpu.sync_copy(xl_rows.at[pl.ds(0, TAIL)],
                        out_sh.at[pl.ds(RPT * NS, TAIL)])

    pltpu.sync_copy(we_hbm, we_v)
    pltpu.sync_copy(att_hbm, att_v)
    we_k = [we_v[pl.ds(k * LANES, LANES)] for k in range(KV)]
    att_k = [att_v[pl.ds(k * LANES, LANES)] for k in range(KV)]

    plsc.subcore_barrier()

    def chunk_body(i, _):
        base = pl.multiple_of(wid * EW + i * CH, 8)
        pltpu.sync_copy(src_hbm.at[pl.ds(base, CH)], idx_s)
        pltpu.sync_copy(dst_hbm.at[pl.ds(base, CH)], idx_d)
        pltpu.sync_copy(attr_hbm.at[pl.ds(base, CH)], attr_v)
        cp1 = pltpu.async_copy(xl_hbm.at[idx_s], xl_rows, sem1)
        cp2 = pltpu.async_copy(xr_hbm.at[idx_d], xr_rows, sem2)
        cp1.wait()
        cp2.wait()

        def group_body(g, _):
            av16 = attr_v[pl.ds(pl.multiple_of(g * LANES, LANES), LANES)]
            eas = zero16
            for j in range(LANES):
                e = g * LANES + j
                av = _take16(av16, _bcast_idx(j))
                acc = zero16
                xls = []
                for k in range(KV):
                    xlv = xl_rows[e, pl.ds(k * LANES, LANES)]
                    xrv = xr_rows[e, pl.ds(k * LANES, LANES)]
                    m = xlv + xrv + av * we_k[k]
                    g_ = jnp.maximum(m, 0.2 * m)
                    acc = acc + g_ * att_k[k]
                    xls.append(xlv)
                eav = jnp.exp(_lanesum(acc))
                for k in range(KV):
                    xl_rows[e, pl.ds(k * LANES, LANES)] = xls[k] * eav
                eas = jnp.where(lane_iota == j, eav, eas)
            ea_buf[pl.ds(pl.multiple_of(g * LANES, LANES), LANES)] = eas
            return 0

        lax.fori_loop(0, NG, group_body, 0)
        pltpu.sync_copy(xl_rows, out_sh.at[idx_d], add=True)
        pltpu.sync_copy(ea_buf, ea_hbm.at[pl.ds(base, CH)])
        return 0

    lax.fori_loop(0, NCHUNK, chunk_body, 0)
    plsc.subcore_barrier()

    for j in range(RPT // ZR):
        r0 = s * RPT + j * ZR
        pltpu.sync_copy(out_sh.at[pl.ds(r0, ZR)], acc_hbm.at[c, pl.ds(r0, ZR)])

    @pl.when(s == NS - 1)
    def _copy_tail():
        pltpu.sync_copy(out_sh.at[pl.ds(RPT * NS, TAIL)],
                        acc_hbm.at[c, pl.ds(RPT * NS, TAIL)])


_msg_pass = pl.kernel(
    _msg_pass_body,
    out_type=(
        jax.ShapeDtypeStruct((NC, N, C), jnp.float32),
        jax.ShapeDtypeStruct((E,), jnp.float32),
    ),
    mesh=plsc.VectorSubcoreMesh(core_axis_name="c", subcore_axis_name="s"),
    scratch_types=(
        pltpu.VMEM((CH,), jnp.int32),
        pltpu.VMEM((CH,), jnp.int32),
        pltpu.VMEM((CH,), jnp.float32),
        pltpu.VMEM((CH, C), jnp.float32),
        pltpu.VMEM((CH, C), jnp.float32),
        pltpu.VMEM((CH,), jnp.float32),
        pltpu.VMEM((C,), jnp.float32),
        pltpu.VMEM((C,), jnp.float32),
        pltpu.VMEM_SHARED((N, C), jnp.float32),
        pltpu.SemaphoreType.DMA,
        pltpu.SemaphoreType.DMA,
    ),
)


def _den_pass_body(H, *refs):
    dst_hbm = refs[0]
    ea_hbms = refs[1:1 + H]
    den_hbm = refs[1 + H]
    idx_d = refs[2 + H]
    ea_vs = refs[3 + H:3 + 2 * H]
    den_rows = refs[3 + 2 * H]
    den_sh = refs[4 + 2 * H]

    c = lax.axis_index("c")
    s = lax.axis_index("s")
    wid = s * NC + c

    zero16 = jnp.zeros((LANES,), jnp.float32)
    lane_iota = lax.iota(jnp.int32, LANES)

    def zrow(r, _):
        for k in range(KV):
            den_rows[r, pl.ds(k * LANES, LANES)] = zero16
        return 0
    lax.fori_loop(0, CH2, zrow, 0)
    for j in range(RPT // ZR):
        r0 = s * RPT + j * ZR
        pltpu.sync_copy(den_rows.at[pl.ds(0, ZR)], den_sh.at[pl.ds(r0, ZR)])

    @pl.when(s == NS - 1)
    def _zero_tail():
        pltpu.sync_copy(den_rows.at[pl.ds(0, TAIL)],
                        den_sh.at[pl.ds(RPT * NS, TAIL)])

    plsc.subcore_barrier()

    def chunk_body(i, _):
        base = pl.multiple_of(wid * EW + i * CH2, 8)
        pltpu.sync_copy(dst_hbm.at[pl.ds(base, CH2)], idx_d)
        for h in range(H):
            pltpu.sync_copy(ea_hbms[h].at[pl.ds(base, CH2)], ea_vs[h])

        def group_body(g, _):
            g16 = pl.multiple_of(g * LANES, LANES)
            eah16 = [ea_vs[h][pl.ds(g16, LANES)] for h in range(H)]
            for j in range(LANES):
                e = g * LANES + j
                row = zero16
                for h in range(H):
                    row = jnp.where(lane_iota == h,
                                    _take16(eah16[h], _bcast_idx(j)), row)
                den_rows[e, pl.ds(0, LANES)] = row
            return 0

        lax.fori_loop(0, NG2, group_body, 0)
        pltpu.sync_copy(den_rows, den_sh.at[idx_d], add=True)
        return 0

    lax.fori_loop(0, NCHUNK2, chunk_body, 0)
    plsc.subcore_barrier()

    for j in range(RPT // ZR):
        r0 = s * RPT + j * ZR
        pltpu.sync_copy(den_sh.at[pl.ds(r0, ZR)], den_hbm.at[c, pl.ds(r0, ZR)])

    @pl.when(s == NS - 1)
    def _copy_tail():
        pltpu.sync_copy(den_sh.at[pl.ds(RPT * NS, TAIL)],
                        den_hbm.at[c, pl.ds(RPT * NS, TAIL)])


def _make_den_pass(H):
    return pl.kernel(
        functools.partial(_den_pass_body, H),
        out_type=jax.ShapeDtypeStruct((NC, N, C), jnp.float32),
        mesh=plsc.VectorSubcoreMesh(core_axis_name="c", subcore_axis_name="s"),
        scratch_types=(
            pltpu.VMEM((CH2,), jnp.int32),
            *[pltpu.VMEM((CH2,), jnp.float32) for _ in range(H)],
            pltpu.VMEM((CH2, C), jnp.float32),
            pltpu.VMEM_SHARED((N, C), jnp.float32),
        ),
    )


_den_pass_by_heads = {4: _make_den_pass(4), 1: _make_den_pass(1)}


BLK = 400


def _front_body(H, x_ref, wl_ref, bl_ref, wr_ref, br_ref, *out_refs):
    xb = x_ref[...]
    xl = jnp.dot(xb, wl_ref[...], preferred_element_type=jnp.float32) + bl_ref[...]
    xr = jnp.dot(xb, wr_ref[...], preferred_element_type=jnp.float32) + br_ref[...]
    for h in range(H):
        out_refs[h][...] = xl[:, h * C:(h + 1) * C]
        out_refs[H + h][...] = xr[:, h * C:(h + 1) * C]


def _front(x, wl, bl, wr, br, H):
    in_d = x.shape[1]
    hc = H * C
    grid = (N // BLK,)
    return pl.pallas_call(
        functools.partial(_front_body, H),
        grid=grid,
        in_specs=[
            pl.BlockSpec((BLK, in_d), lambda i: (i, 0)),
            pl.BlockSpec((in_d, hc), lambda i: (0, 0)),
            pl.BlockSpec((1, hc), lambda i: (0, 0)),
            pl.BlockSpec((in_d, hc), lambda i: (0, 0)),
            pl.BlockSpec((1, hc), lambda i: (0, 0)),
        ],
        out_specs=[pl.BlockSpec((BLK, C), lambda i: (i, 0))] * (2 * H),
        out_shape=[jax.ShapeDtypeStruct((N, C), jnp.float32)] * (2 * H),
    )(x, wl, bl.reshape(1, hc), wr, br.reshape(1, hc))


def _epilogue_body(H, has_proj, eff, x_ref, bias_ref, gw_ref, gb_ref,
                   w1_ref, b1_ref, w2_ref, b2_ref, lng_ref, lnb_ref,
                   *rest):
    if has_proj:
        pw_ref, pb_ref = rest[0], rest[1]
        rest = rest[2:]
    acc_refs = rest[:H]
    den_refs = rest[H:H + 1]
    y_ref = rest[H + 1]

    den_ref = den_refs[0]
    cols = []
    for h in range(H):
        acc = acc_refs[h][0] + acc_refs[h][1]
        den = den_ref[0, :, h:h + 1] + den_ref[1, :, h:h + 1]
        cols.append(acc * (1.0 / (den + 1e-16)))
    hfull = cols[0] if H == 1 else jnp.concatenate(cols, axis=1)
    hfull = hfull + bias_ref[...]
    hact = jnp.where(hfull > 0, hfull, jnp.exp(jnp.minimum(hfull, 0.0)) - 1.0)
    gate = jax.nn.sigmoid(
        jnp.dot(hact, gw_ref[...], preferred_element_type=jnp.float32) + gb_ref[0, 0])
    xb = x_ref[...]
    if has_proj:
        xp = jnp.dot(xb, pw_ref[...], preferred_element_type=jnp.float32) + pb_ref[...]
    else:
        xp = xb
    y = gate * hact + (1.0 - gate) * xp
    f = jnp.dot(jnp.maximum(
        jnp.dot(y, w1_ref[...], preferred_element_type=jnp.float32) + b1_ref[...],
        0.0), w2_ref[...], preferred_element_type=jnp.float32) + b2_ref[...]
    y = y + f
    mu = jnp.mean(y, axis=-1, keepdims=True)
    var = jnp.mean((y - mu) ** 2, axis=-1, keepdims=True)
    y_ref[...] = (y - mu) * jax.lax.rsqrt(var + 1e-5) * lng_ref[...] + lnb_ref[...]


def _epilogue(x, p, H, acc_list, den_list):
    in_d = x.shape[1]
    eff = H * C
    has_proj = 'proj_W' in p
    grid = (N // BLK,)
    in_specs = [
        pl.BlockSpec((BLK, in_d), lambda i: (i, 0)),
        pl.BlockSpec((1, eff), lambda i: (0, 0)),
        pl.BlockSpec((eff, 1), lambda i: (0, 0)),
        pl.BlockSpec((1, 1), lambda i: (0, 0)),
        pl.BlockSpec((eff, 2 * eff), lambda i: (0, 0)),
        pl.BlockSpec((1, 2 * eff), lambda i: (0, 0)),
        pl.BlockSpec((2 * eff, eff), lambda i: (0, 0)),
        pl.BlockSpec((1, eff), lambda i: (0, 0)),
        pl.BlockSpec((1, eff), lambda i: (0, 0)),
        pl.BlockSpec((1, eff), lambda i: (0, 0)),
    ]
    args = [x, p['bias'].reshape(1, eff), p['gate_W'],
            p['gate_b'].reshape(1, 1), p['ffn_W1'], p['ffn_b1'].reshape(1, 2 * eff),
            p['ffn_W2'], p['ffn_b2'].reshape(1, eff),
            p['ln_g'].reshape(1, eff), p['ln_b'].reshape(1, eff)]
    if has_proj:
        in_specs += [
            pl.BlockSpec((in_d, eff), lambda i: (0, 0)),
            pl.BlockSpec((1, eff), lambda i: (0, 0)),
        ]
        args += [p['proj_W'], p['proj_b'].reshape(1, eff)]
    in_specs += [pl.BlockSpec((NC, BLK, C), lambda i: (0, i, 0))] * (H + 1)
    args += acc_list + den_list
    return pl.pallas_call(
        functools.partial(_epilogue_body, H, has_proj, eff),
        grid=grid,
        in_specs=in_specs,
        out_specs=pl.BlockSpec((BLK, eff), lambda i: (i, 0)),
        out_shape=jax.ShapeDtypeStruct((N, eff), jnp.float32),
    )(*args)


def _block(x, src, dst, attr, p, H):
    fronts = _front(x, p['Wl'], p['bl'], p['Wr'], p['br'], H)
    xls, xrs = fronts[:H], fronts[H:]
    acc_list, ea_list = [], []
    for h in range(H):
        we_h = p['We'][0, h * C:(h + 1) * C]
        att_h = p['att'][h]
        acc, ea = _msg_pass(xls[h], xrs[h], src, dst, attr, we_h, att_h)
        acc_list.append(acc)
        ea_list.append(ea)
    den = _den_pass_by_heads[H](dst, *ea_list)
    return _epilogue(x, p, H, acc_list, [den])


def kernel(x, edge_index, edge_attr, params):
    src = edge_index[0].astype(jnp.int32)
    dst = edge_index[1].astype(jnp.int32)
    attr = edge_attr.reshape(E).astype(jnp.float32)
    h = x
    for p, H in zip(params, HEADS):
        h = _block(h, src, dst, attr, p, H)
    return h
